# baseline (speedup 1.0000x reference)
import functools

import jax
import jax.numpy as jnp
from jax import lax
from jax.experimental import pallas as pl
from jax.experimental.layout import Format, Layout
from jax.experimental.pallas import tpu as pltpu
from jax.experimental.pallas import tpu_sc as plsc

N_ROWS = 1_000_000
DIM = 64
BATCH = 819_200

NUM_CORES = 2
NUM_SUBCORES = 16
LANES = 16
NUM_WORKERS = NUM_CORES * NUM_SUBCORES

B_PER_W = BATCH // NUM_WORKERS
CHUNK = 256
SUB = 128
KSUB = CHUNK // SUB
NUM_CHUNKS = B_PER_W // CHUNK


def _sampler_body(table_hbm, param_hbm, out_hbm,
                  param_v0, param_v1, w_v0, w_v1,
                  idx0_v0, idx0_v1, idx1_v0, idx1_v1,
                  rows0_v0, rows0_v1, rows1_v0, rows1_v1,
                  sem_g0, sem_g1, sem_p0, sem_p1, sem_o0, sem_o1):
    wid = lax.axis_index("s") * NUM_CORES + lax.axis_index("c")
    base = wid * B_PER_W
    scale = jnp.float32(N_ROWS - 1)

    param_v = [param_v0, param_v1]
    w_v = [w_v0, w_v1]
    idx0_v = [idx0_v0, idx0_v1]
    idx1_v = [idx1_v0, idx1_v1]
    rows0_v = [rows0_v0, rows0_v1]
    rows1_v = [rows1_v0, rows1_v1]
    sem_g = [sem_g0, sem_g1]
    sem_p = [sem_p0, sem_p1]
    sem_o = [sem_o0, sem_o1]

    def compute_idx(s):
        for j in range(CHUNK // LANES):
            p = param_v[s][pl.ds(j * LANES, LANES)]
            t = jnp.minimum(jnp.maximum(p, 0.0), 1.0) * scale
            i0 = t.astype(jnp.int32)
            i1 = jnp.minimum(i0 + 1, N_ROWS - 1)
            w = t - i0.astype(jnp.float32)
            k, r = divmod(j * LANES, SUB)
            idx0_v[s][k, pl.ds(r, LANES)] = i0
            idx1_v[s][k, pl.ds(r, LANES)] = i1
            w_v[s][pl.ds(j * LANES, LANES)] = w

    def issue_gathers(s):
        for k in range(KSUB):
            pltpu.async_copy(table_hbm.at[idx0_v[s].at[k]],
                             rows0_v[s].at[k], sem_g[s])
            pltpu.async_copy(table_hbm.at[idx1_v[s].at[k]],
                             rows1_v[s].at[k], sem_g[s])

    def wait_gathers(s):
        for k in range(KSUB):
            pltpu.make_async_copy(table_hbm.at[pl.ds(0, SUB), :],
                                  rows0_v[s].at[k], sem_g[s]).wait()
            pltpu.make_async_copy(table_hbm.at[pl.ds(0, SUB), :],
                                  rows1_v[s].at[k], sem_g[s]).wait()

    def issue_param(g, s):
        off = pl.multiple_of(base + g * CHUNK, CHUNK)
        pltpu.async_copy(param_hbm.at[pl.ds(off, CHUNK)], param_v[s],
                         sem_p[s])

    def wait_param(s):
        pltpu.make_async_copy(param_hbm.at[pl.ds(0, CHUNK)], param_v[s],
                              sem_p[s]).wait()

    def lerp(s):
        for k in range(KSUB):
            def row16(r16, c, _k=k):
                w16 = w_v[s][pl.ds(_k * SUB + r16 * LANES, LANES)]
                for j in range(LANES):
                    wb = w16.at[jnp.full((LANES,), j, jnp.int32)].get(
                        mode="promise_in_bounds")
                    one_m = 1.0 - wb
                    r = r16 * LANES + j
                    for cc in range(DIM // LANES):
                        v0 = rows0_v[s][_k, r, pl.ds(cc * LANES, LANES)]
                        v1 = rows1_v[s][_k, r, pl.ds(cc * LANES, LANES)]
                        rows0_v[s][_k, r, pl.ds(cc * LANES, LANES)] = (
                            v0 * one_m + v1 * wb)
                return c
            lax.fori_loop(0, SUB // LANES, row16, 0)

    def issue_out(g, s):
        off = pl.multiple_of(base + g * CHUNK, CHUNK)
        for k in range(KSUB):
            pltpu.async_copy(rows0_v[s].at[k],
                             out_hbm.at[pl.ds(off + k * SUB, SUB)], sem_o[s])

    def wait_out(s):
        for k in range(KSUB):
            pltpu.make_async_copy(rows0_v[s].at[k],
                                  out_hbm.at[pl.ds(0, SUB)], sem_o[s]).wait()

    issue_param(0, 0)
    wait_param(0)
    compute_idx(0)
    issue_gathers(0)
    issue_param(1, 1)

    def half(g, cur, nxt):
        @pl.when(g + 1 < NUM_CHUNKS)
        def _():
            wait_param(nxt)
            compute_idx(nxt)

            @pl.when(g + 1 >= 2)
            def _():
                wait_out(nxt)
            issue_gathers(nxt)

        @pl.when(g + 2 < NUM_CHUNKS)
        def _():
            issue_param(g + 2, cur)

        wait_gathers(cur)
        lerp(cur)
        issue_out(g, cur)

    def body(i, carry):
        half(2 * i, 0, 1)
        half(2 * i + 1, 1, 0)
        return carry

    lax.fori_loop(0, NUM_CHUNKS // 2, body, 0)
    wait_out(0)
    wait_out(1)


_OUT_FMT = Format(Layout((1, 0)),
                  jax.sharding.SingleDeviceSharding(jax.devices()[0]))


@functools.partial(jax.jit, out_shardings=_OUT_FMT)
def kernel(input, param):
    mesh = plsc.VectorSubcoreMesh(core_axis_name="c", subcore_axis_name="s")
    f = pl.kernel(
        _sampler_body,
        out_type=jax.ShapeDtypeStruct((BATCH, DIM), jnp.float32),
        mesh=mesh,
        scratch_types=[
            pltpu.VMEM((CHUNK,), jnp.float32),
            pltpu.VMEM((CHUNK,), jnp.float32),
            pltpu.VMEM((CHUNK,), jnp.float32),
            pltpu.VMEM((CHUNK,), jnp.float32),
            pltpu.VMEM((KSUB, SUB), jnp.int32),
            pltpu.VMEM((KSUB, SUB), jnp.int32),
            pltpu.VMEM((KSUB, SUB), jnp.int32),
            pltpu.VMEM((KSUB, SUB), jnp.int32),
            pltpu.VMEM((KSUB, SUB, DIM), jnp.float32),
            pltpu.VMEM((KSUB, SUB, DIM), jnp.float32),
            pltpu.VMEM((KSUB, SUB, DIM), jnp.float32),
            pltpu.VMEM((KSUB, SUB, DIM), jnp.float32),
            pltpu.SemaphoreType.DMA,
            pltpu.SemaphoreType.DMA,
            pltpu.SemaphoreType.DMA,
            pltpu.SemaphoreType.DMA,
            pltpu.SemaphoreType.DMA,
            pltpu.SemaphoreType.DMA,
        ],
        compiler_params=pltpu.CompilerParams(use_tc_tiling_on_sc=False),
    )(input, param)
    return f


# speedup vs baseline: 1.0008x; 1.0008x over previous
import functools

import jax
import jax.numpy as jnp
from jax import lax
from jax.experimental import pallas as pl
from jax.experimental.layout import Format, Layout
from jax.experimental.pallas import tpu as pltpu
from jax.experimental.pallas import tpu_sc as plsc

N_ROWS = 1_000_000
DIM = 64
BATCH = 819_200

NUM_CORES = 2
NUM_SUBCORES = 16
LANES = 16
NUM_WORKERS = NUM_CORES * NUM_SUBCORES

B_PER_W = BATCH // NUM_WORKERS
CHUNK = 256
SUB = 128
KSUB = CHUNK // SUB
NUM_CHUNKS = B_PER_W // CHUNK


def _sampler_body(table_hbm, param_hbm, out_hbm,
                  param_v0, param_v1, w_v0, w_v1,
                  idx0_v0, idx0_v1, idx1_v0, idx1_v1,
                  rows0_v0, rows0_v1, rows1_v0, rows1_v1,
                  sem_g0, sem_g1, sem_p0, sem_p1, sem_o0, sem_o1):
    wid = lax.axis_index("s") * NUM_CORES + lax.axis_index("c")
    base = wid * B_PER_W
    scale = jnp.float32(N_ROWS - 1)

    param_v = [param_v0, param_v1]
    w_v = [w_v0, w_v1]
    idx0_v = [idx0_v0, idx0_v1]
    idx1_v = [idx1_v0, idx1_v1]
    rows0_v = [rows0_v0, rows0_v1]
    rows1_v = [rows1_v0, rows1_v1]
    sem_g = [sem_g0, sem_g1]
    sem_p = [sem_p0, sem_p1]
    sem_o = [sem_o0, sem_o1]

    def compute_idx(s):
        for j in range(CHUNK // LANES):
            p = param_v[s][pl.ds(j * LANES, LANES)]
            t = jnp.minimum(jnp.maximum(p, 0.0), 1.0) * scale
            i0 = t.astype(jnp.int32)
            i1 = jnp.minimum(i0 + 1, N_ROWS - 1)
            w = t - i0.astype(jnp.float32)
            k, r = divmod(j * LANES, SUB)
            idx0_v[s][k, pl.ds(r, LANES)] = i0
            idx1_v[s][k, pl.ds(r, LANES)] = i1
            w_v[s][pl.ds(j * LANES, LANES)] = w

    def issue_gathers(s):
        for k in range(KSUB):
            pltpu.async_copy(table_hbm.at[idx0_v[s].at[k]],
                             rows0_v[s].at[k], sem_g[s])
            pltpu.async_copy(table_hbm.at[idx1_v[s].at[k]],
                             rows1_v[s].at[k], sem_g[s])

    def wait_gathers(s):
        for k in range(KSUB):
            pltpu.make_async_copy(table_hbm.at[pl.ds(0, SUB), :],
                                  rows0_v[s].at[k], sem_g[s]).wait()
            pltpu.make_async_copy(table_hbm.at[pl.ds(0, SUB), :],
                                  rows1_v[s].at[k], sem_g[s]).wait()

    def issue_param(g, s):
        off = pl.multiple_of(base + g * CHUNK, CHUNK)
        pltpu.async_copy(param_hbm.at[pl.ds(off, CHUNK)], param_v[s],
                         sem_p[s])

    def wait_param(s):
        pltpu.make_async_copy(param_hbm.at[pl.ds(0, CHUNK)], param_v[s],
                              sem_p[s]).wait()

    def lerp(s):
        for k in range(KSUB):
            def row16(r16, c, _k=k):
                w16 = w_v[s][pl.ds(_k * SUB + r16 * LANES, LANES)]
                for j in range(LANES):
                    wb = w16.at[jnp.full((LANES,), j, jnp.int32)].get(
                        mode="promise_in_bounds")
                    one_m = 1.0 - wb
                    r = r16 * LANES + j
                    for cc in range(DIM // LANES):
                        v0 = rows0_v[s][_k, r, pl.ds(cc * LANES, LANES)]
                        v1 = rows1_v[s][_k, r, pl.ds(cc * LANES, LANES)]
                        rows0_v[s][_k, r, pl.ds(cc * LANES, LANES)] = (
                            v0 * one_m + v1 * wb)
                return c
            lax.fori_loop(0, SUB // LANES, row16, 0)

    def issue_out(g, s):
        off = pl.multiple_of(base + g * CHUNK, CHUNK)
        for k in range(KSUB):
            pltpu.async_copy(rows0_v[s].at[k],
                             out_hbm.at[pl.ds(off + k * SUB, SUB)], sem_o[s])

    def wait_out(s):
        for k in range(KSUB):
            pltpu.make_async_copy(rows0_v[s].at[k],
                                  out_hbm.at[pl.ds(0, SUB)], sem_o[s]).wait()

    issue_param(0, 0)
    wait_param(0)
    compute_idx(0)
    issue_gathers(0)
    issue_param(1, 1)

    def half(g, cur, nxt):
        @pl.when(g + 1 < NUM_CHUNKS)
        def _():
            wait_param(nxt)
            compute_idx(nxt)

            @pl.when(g + 1 >= 2)
            def _():
                wait_out(nxt)
            issue_gathers(nxt)

        @pl.when(g + 2 < NUM_CHUNKS)
        def _():
            issue_param(g + 2, cur)

        wait_gathers(cur)
        lerp(cur)
        issue_out(g, cur)

    def body(i, carry):
        half(2 * i, 0, 1)
        half(2 * i + 1, 1, 0)
        return carry

    lax.fori_loop(0, NUM_CHUNKS // 2, body, 0)
    wait_out(0)
    wait_out(1)


_OUT_FMT = Format(Layout((1, 0), tiling=()),
                  jax.sharding.SingleDeviceSharding(jax.devices()[0]))


@functools.partial(jax.jit, out_shardings=_OUT_FMT)
def kernel(input, param):
    mesh = plsc.VectorSubcoreMesh(core_axis_name="c", subcore_axis_name="s")
    f = pl.kernel(
        _sampler_body,
        out_type=jax.ShapeDtypeStruct((BATCH, DIM), jnp.float32),
        mesh=mesh,
        scratch_types=[
            pltpu.VMEM((CHUNK,), jnp.float32),
            pltpu.VMEM((CHUNK,), jnp.float32),
            pltpu.VMEM((CHUNK,), jnp.float32),
            pltpu.VMEM((CHUNK,), jnp.float32),
            pltpu.VMEM((KSUB, SUB), jnp.int32),
            pltpu.VMEM((KSUB, SUB), jnp.int32),
            pltpu.VMEM((KSUB, SUB), jnp.int32),
            pltpu.VMEM((KSUB, SUB), jnp.int32),
            pltpu.VMEM((KSUB, SUB, DIM), jnp.float32),
            pltpu.VMEM((KSUB, SUB, DIM), jnp.float32),
            pltpu.VMEM((KSUB, SUB, DIM), jnp.float32),
            pltpu.VMEM((KSUB, SUB, DIM), jnp.float32),
            pltpu.SemaphoreType.DMA,
            pltpu.SemaphoreType.DMA,
            pltpu.SemaphoreType.DMA,
            pltpu.SemaphoreType.DMA,
            pltpu.SemaphoreType.DMA,
            pltpu.SemaphoreType.DMA,
        ],
        compiler_params=pltpu.CompilerParams(use_tc_tiling_on_sc=False),
    )(input, param)
    return f
